# baseline (speedup 1.0000x reference)
import functools

import jax
import jax.numpy as jnp
from jax import lax
from jax.experimental import pallas as pl
from jax.experimental.pallas import tpu as pltpu
from jax.experimental.pallas import tpu_sc as plsc

_NC, _NS, _LANES = 2, 16, 16
_NW = _NC * _NS

_RBLK = 32


def _sc_add(B, S, D):
    mesh = plsc.VectorSubcoreMesh(core_axis_name="c", subcore_axis_name="s")
    s_per_w = S // _NW
    n_blk = s_per_w // _RBLK
    n_chunk = D // _LANES

    @functools.partial(
        pl.kernel,
        out_type=jax.ShapeDtypeStruct((B * S, D), jnp.float32),
        mesh=mesh,
        scratch_types=[
            pltpu.VMEM((_RBLK, D), jnp.float32),
            pltpu.VMEM((_RBLK, D), jnp.float32),
        ],
    )
    def k(in_hbm, emb_hbm, out_hbm, emb_buf, in_buf):
        wid = lax.axis_index("s") * _NC + lax.axis_index("c")
        s0 = wid * s_per_w

        def blk_body(blk, carry):
            r0 = s0 + blk * _RBLK
            pltpu.sync_copy(emb_hbm.at[pl.ds(r0, _RBLK), :], emb_buf)

            def b_body(b, carry2):
                row0 = b * S + r0
                pltpu.sync_copy(in_hbm.at[pl.ds(row0, _RBLK), :], in_buf)

                def row_body(i, c3):
                    for j in range(n_chunk):
                        e = emb_buf[i, pl.ds(j * _LANES, _LANES)]
                        plsc.addupdate(in_buf.at[i, pl.ds(j * _LANES, _LANES)], e)
                    return c3

                lax.fori_loop(0, _RBLK, row_body, 0, unroll=False)
                pltpu.sync_copy(in_buf, out_hbm.at[pl.ds(row0, _RBLK), :])
                return carry2

            lax.fori_loop(0, B, b_body, 0, unroll=False)
            return carry

        lax.fori_loop(0, n_blk, blk_body, 0, unroll=False)

    return k


def kernel(inputs, embeddings):
    B, S, D = inputs.shape
    pos = embeddings[:S]
    out = _sc_add(B, S, D)(inputs.reshape(B * S, D), pos)
    return out.reshape(B, S, D)


# speedup vs baseline: 3.9905x; 3.9905x over previous
import jax
import jax.numpy as jnp
from jax.experimental import pallas as pl


def _add_body(x_ref, e_ref, o_ref):
    o_ref[...] = x_ref[...] + e_ref[...]


def kernel(inputs, embeddings):
    B, S, D = inputs.shape
    pos = embeddings[:S]
    SBLK = 512
    grid = (S // SBLK,)
    return pl.pallas_call(
        _add_body,
        grid=grid,
        in_specs=[
            pl.BlockSpec((B, SBLK, D), lambda i: (0, i, 0)),
            pl.BlockSpec((SBLK, D), lambda i: (i, 0)),
        ],
        out_specs=pl.BlockSpec((B, SBLK, D), lambda i: (0, i, 0)),
        out_shape=jax.ShapeDtypeStruct((B, S, D), inputs.dtype),
    )(inputs, pos)
